# Initial kernel scaffold; baseline (speedup 1.0000x reference)
import jax
import jax.numpy as jnp
from jax.experimental import pallas as pl


def kernel(f_idx, emb_f):
    raise NotImplementedError("write your pallas kernel here")


# speedup vs baseline: 4.4601x; 4.4601x over previous
import functools

import jax
import jax.numpy as jnp
from jax import lax
from jax.experimental import pallas as pl
from jax.experimental.pallas import tpu as pltpu
from jax.experimental.pallas import tpu_sc as plsc

_NC = 2
_NS = 16
_NW = _NC * _NS

_B = 16384 * 100
_D = 64
_C = 128
_B_PER_W = _B // _NW
_CHUNKS = _B_PER_W // _C


def _body(idx_hbm, table_hbm, out_hbm, idx_v, rows_v, sem):
    wid = lax.axis_index("s") * _NC + lax.axis_index("c")
    base = wid * _B_PER_W

    def chunk(g, _):
        off = base + g * _C
        pltpu.sync_copy(idx_hbm.at[pl.ds(off, _C)], idx_v)
        pltpu.async_copy(table_hbm.at[idx_v], rows_v, sem).wait()
        pltpu.sync_copy(rows_v, out_hbm.at[pl.ds(off, _C)])
        return _

    lax.fori_loop(0, _CHUNKS, chunk, None)


def kernel(f_idx, emb_f):
    idx_flat = f_idx.reshape(_B).astype(jnp.int32)
    run = pl.kernel(
        _body,
        mesh=plsc.VectorSubcoreMesh(core_axis_name="c", subcore_axis_name="s"),
        out_type=jax.ShapeDtypeStruct((_B, _D), jnp.float32),
        scratch_types=[
            pltpu.VMEM((_C,), jnp.int32),
            pltpu.VMEM((_C, _D), jnp.float32),
            pltpu.SemaphoreType.DMA,
        ],
        compiler_params=pltpu.CompilerParams(use_tc_tiling_on_sc=False),
    )
    out = run(idx_flat, emb_f)
    return out.reshape(f_idx.shape[0], f_idx.shape[1], _D)


# speedup vs baseline: 4.5326x; 1.0163x over previous
import jax
import jax.numpy as jnp
from jax import lax
from jax.experimental import pallas as pl
from jax.experimental.pallas import tpu as pltpu
from jax.experimental.pallas import tpu_sc as plsc

_NC = 2
_NS = 16
_NW = _NC * _NS

_B = 16384 * 100
_D = 64
_C = 128
_NBUF = 4
_B_PER_W = _B // _NW
_CHUNKS = _B_PER_W // _C
_ROUNDS = _CHUNKS // _NBUF


def _body(idx_hbm, table_hbm, out_hbm, idx_v, rows_v, gsem, ssem):
    wid = lax.axis_index("s") * _NC + lax.axis_index("c")
    base = wid * _B_PER_W
    pltpu.sync_copy(idx_hbm.at[pl.ds(base, _B_PER_W)], idx_v)

    def gather(g, b):
        return pltpu.make_async_copy(
            table_hbm.at[idx_v.at[pl.ds(g * _C, _C)]], rows_v.at[b], gsem.at[b]
        )

    def scatter(g, b):
        return pltpu.make_async_copy(
            rows_v.at[b], out_hbm.at[pl.ds(base + g * _C, _C)], ssem.at[b]
        )

    for b in range(_NBUF):
        gather(b, b).start()

    def round_body(r, carry):
        for b in range(_NBUF):
            g = r * _NBUF + b
            gather(g, b).wait()
            scatter(g, b).start()

            @pl.when(r < _ROUNDS - 1)
            def _refill():
                scatter(g, b).wait()
                gather(g + _NBUF, b).start()

        return carry

    lax.fori_loop(0, _ROUNDS, round_body, 0)
    for b in range(_NBUF):
        scatter((_ROUNDS - 1) * _NBUF + b, b).wait()


def kernel(f_idx, emb_f):
    idx_flat = f_idx.reshape(_B).astype(jnp.int32)
    run = pl.kernel(
        _body,
        mesh=plsc.VectorSubcoreMesh(core_axis_name="c", subcore_axis_name="s"),
        out_type=jax.ShapeDtypeStruct((_B, _D), jnp.float32),
        scratch_types=[
            pltpu.VMEM((_B_PER_W,), jnp.int32),
            pltpu.VMEM((_NBUF, _C, _D), jnp.float32),
            pltpu.SemaphoreType.DMA((_NBUF,)),
            pltpu.SemaphoreType.DMA((_NBUF,)),
        ],
        compiler_params=pltpu.CompilerParams(use_tc_tiling_on_sc=False),
    )
    out = run(idx_flat, emb_f)
    return out.reshape(f_idx.shape[0], f_idx.shape[1], _D)


# speedup vs baseline: 13.5147x; 2.9817x over previous
import jax
import jax.numpy as jnp
from jax import lax
from jax.experimental import pallas as pl
from jax.experimental.pallas import tpu as pltpu
from jax.experimental.pallas import tpu_sc as plsc

_NC = 2
_NS = 16
_NW = _NC * _NS

_B = 16384 * 100
_D = 64
_C = 128
_NBUF = 4
_B_PER_W = _B // _NW
_CHUNKS = _B_PER_W // _C
_ROUNDS = _CHUNKS // _NBUF


def _body(idx_hbm, table_hbm, out_hbm, idx_v, rows_v, table_sh, gsem, ssem):
    wid = lax.axis_index("s") * _NC + lax.axis_index("c")
    base = wid * _B_PER_W

    @pl.when(lax.axis_index("s") == 0)
    def _stage_table():
        pltpu.sync_copy(table_hbm, table_sh)

    plsc.subcore_barrier()
    pltpu.sync_copy(idx_hbm.at[pl.ds(base, _B_PER_W)], idx_v)

    def gather(g, b):
        return pltpu.make_async_copy(
            table_sh.at[idx_v.at[pl.ds(g * _C, _C)]], rows_v.at[b], gsem.at[b]
        )

    def scatter(g, b):
        return pltpu.make_async_copy(
            rows_v.at[b], out_hbm.at[pl.ds(base + g * _C, _C)], ssem.at[b]
        )

    for b in range(_NBUF):
        gather(b, b).start()

    def round_body(r, carry):
        for b in range(_NBUF):
            g = r * _NBUF + b
            gather(g, b).wait()
            scatter(g, b).start()

            @pl.when(r < _ROUNDS - 1)
            def _refill():
                scatter(g, b).wait()
                gather(g + _NBUF, b).start()

        return carry

    lax.fori_loop(0, _ROUNDS, round_body, 0)
    for b in range(_NBUF):
        scatter((_ROUNDS - 1) * _NBUF + b, b).wait()


def kernel(f_idx, emb_f):
    idx_flat = f_idx.reshape(_B).astype(jnp.int32)
    run = pl.kernel(
        _body,
        mesh=plsc.VectorSubcoreMesh(core_axis_name="c", subcore_axis_name="s"),
        out_type=jax.ShapeDtypeStruct((_B, _D), jnp.float32),
        scratch_types=[
            pltpu.VMEM((_B_PER_W,), jnp.int32),
            pltpu.VMEM((_NBUF, _C, _D), jnp.float32),
            pltpu.VMEM_SHARED((34, _D), jnp.float32),
            pltpu.SemaphoreType.DMA((_NBUF,)),
            pltpu.SemaphoreType.DMA((_NBUF,)),
        ],
        compiler_params=pltpu.CompilerParams(use_tc_tiling_on_sc=False),
    )
    out = run(idx_flat, emb_f)
    return out.reshape(f_idx.shape[0], f_idx.shape[1], _D)
